# Initial kernel scaffold; baseline (speedup 1.0000x reference)
import jax
import jax.numpy as jnp
from jax.experimental import pallas as pl


def kernel(x, y, cfc1_weight, afc1_weight):
    raise NotImplementedError("write your pallas kernel here")


# speedup vs baseline: 3.9887x; 3.9887x over previous
import functools

import jax
import jax.numpy as jnp
from jax import lax
from jax.experimental import pallas as pl
from jax.experimental.pallas import tpu as pltpu
from jax.experimental.pallas import tpu_sc as plsc

B = 4096
DIM = 128
LANES = 16
NUM_WORKERS = 32
B_PER_W = B // NUM_WORKERS
CHUNKS = DIM // LANES


def _body(cfc1_hbm, x_hbm, afc1_hbm, y_hbm, out_hbm,
          idx_x, idx_y, rows_x, rows_y, pbuf, out_v, sem_x, sem_y):
    num_cores = 2
    wid = lax.axis_index("s") * num_cores + lax.axis_index("c")
    base = wid * B_PER_W

    pltpu.sync_copy(x_hbm.at[pl.ds(base, B_PER_W)], idx_x)
    pltpu.sync_copy(y_hbm.at[pl.ds(base, B_PER_W)], idx_y)

    cpx = pltpu.async_copy(cfc1_hbm.at[idx_x], rows_x, sem_x)
    cpy = pltpu.async_copy(afc1_hbm.at[idx_y], rows_y, sem_y)
    cpx.wait()
    cpy.wait()

    def row(r, _):
        acc = rows_x[r, pl.ds(0, LANES)] * rows_y[r, pl.ds(0, LANES)]
        for c in range(1, CHUNKS):
            acc = acc + (rows_x[r, pl.ds(c * LANES, LANES)]
                         * rows_y[r, pl.ds(c * LANES, LANES)])
        pbuf[pl.ds(r * (LANES + 1), LANES)] = acc
        return _

    lax.fori_loop(0, B_PER_W, row, 0)

    lane = lax.iota(jnp.int32, LANES)

    def group(g, _):
        flat = (g * LANES + lane) * (LANES + 1)
        acc = plsc.load_gather(pbuf, [flat])
        for k in range(1, LANES):
            acc = acc + plsc.load_gather(pbuf, [flat + k])
        out_v[pl.ds(g * LANES, LANES)] = acc
        return _

    lax.fori_loop(0, B_PER_W // LANES, group, 0)

    pltpu.sync_copy(out_v, out_hbm.at[pl.ds(base, B_PER_W)])


@jax.jit
def kernel(x, y, cfc1_weight, afc1_weight):
    mesh = plsc.VectorSubcoreMesh(core_axis_name="c", subcore_axis_name="s")
    run = pl.kernel(
        _body,
        out_type=jax.ShapeDtypeStruct((B,), jnp.float32),
        mesh=mesh,
        scratch_types=[
            pltpu.VMEM((B_PER_W,), jnp.int32),
            pltpu.VMEM((B_PER_W,), jnp.int32),
            pltpu.VMEM((B_PER_W, DIM), jnp.float32),
            pltpu.VMEM((B_PER_W, DIM), jnp.float32),
            pltpu.VMEM((B_PER_W * (LANES + 1),), jnp.float32),
            pltpu.VMEM((B_PER_W,), jnp.float32),
            pltpu.SemaphoreType.DMA,
            pltpu.SemaphoreType.DMA,
        ],
        compiler_params=pltpu.CompilerParams(needs_layout_passes=False),
    )
    return run(cfc1_weight, x, afc1_weight, y)
